# baseline (speedup 1.0000x reference)
import functools

import jax
import jax.numpy as jnp
from jax import lax
from jax.experimental import pallas as pl
from jax.experimental.pallas import tpu as pltpu, tpu_sc as plsc

VOCAB = 49408
HIDDEN = 512
MAX_POS = 77
NC = 2
NS = 16
NW = NC * NS
LANES = 16
CH_ROWS = 56
CHUNKS = 44


def _sc_embed(ids3, table, pe):
    mesh = plsc.VectorSubcoreMesh(core_axis_name="c", subcore_axis_name="s")

    @functools.partial(
        pl.kernel,
        mesh=mesh,
        out_type=jax.ShapeDtypeStruct(
            (NW, CHUNKS, CH_ROWS, HIDDEN), jnp.float32),
        scratch_types=[
            pltpu.VMEM((CHUNKS, CH_ROWS), jnp.int32),
            pltpu.VMEM((MAX_POS, HIDDEN), jnp.float32),
            pltpu.VMEM((CH_ROWS, HIDDEN), jnp.float32),
            pltpu.VMEM((CH_ROWS, HIDDEN), jnp.float32),
            pltpu.SemaphoreType.DMA,
            pltpu.SemaphoreType.DMA,
        ],
    )
    def k(ids_hbm, table_hbm, pe_hbm, out_hbm, idx_v, pe_v, rows0, rows1,
          sem0, sem1):
        wid = lax.axis_index("s") * NC + lax.axis_index("c")
        pltpu.sync_copy(ids_hbm.at[wid], idx_v)
        pltpu.sync_copy(pe_hbm, pe_v)
        rows = (rows0, rows1)
        sems = (sem0, sem1)

        pltpu.async_copy(table_hbm.at[idx_v.at[0]], rows[0], sems[0])

        def pair_body(i, carry):
            for b in range(2):
                c = 2 * i + b
                other = 1 - b
                pltpu.make_async_copy(
                    table_hbm.at[idx_v.at[c]], rows[b], sems[b]).wait()

                @pl.when(c + 1 < CHUNKS)
                def _():
                    pltpu.async_copy(
                        table_hbm.at[idx_v.at[c + 1]], rows[other],
                        sems[other])

                p0 = lax.rem(c * CH_ROWS, MAX_POS)

                @plsc.parallel_loop(0, CH_ROWS, step=1, unroll=4)
                def _(r):
                    p = r + p0
                    p = jnp.where(p >= MAX_POS, p - MAX_POS, p)
                    for j in range(HIDDEN // LANES):
                        sl = pl.ds(j * LANES, LANES)
                        plsc.addupdate(rows[b].at[r, sl], pe_v[p, sl])

                pltpu.sync_copy(rows[b], out_hbm.at[wid, c])
            return carry

        lax.fori_loop(0, CHUNKS // 2, pair_body, 0, unroll=1)

    return k(ids3, table, pe)


def kernel(input_ids, token_table, pos_emb):
    batch, seq = input_ids.shape
    ids3 = input_ids.astype(jnp.int32).reshape(NW, CHUNKS, CH_ROWS)
    pe = pos_emb.astype(jnp.float32).reshape(MAX_POS, HIDDEN)
    out = _sc_embed(ids3, token_table.astype(jnp.float32), pe)
    return out.reshape(batch, seq, HIDDEN)


# speedup vs baseline: 2.7961x; 2.7961x over previous
import functools

import jax
import jax.numpy as jnp
from jax import lax
from jax.experimental import pallas as pl
from jax.experimental.pallas import tpu as pltpu, tpu_sc as plsc

VOCAB = 49408
HIDDEN = 512
MAX_POS = 77
NC = 2
NS = 16
NW = NC * NS
LANES = 16
NB = 32


def _sc_embed(ids3, table, pe):
    mesh = plsc.VectorSubcoreMesh(core_axis_name="c", subcore_axis_name="s")

    @functools.partial(
        pl.kernel,
        mesh=mesh,
        out_type=jax.ShapeDtypeStruct(
            (MAX_POS, NW * NB, HIDDEN), jnp.float32),
        scratch_types=[
            pltpu.VMEM((MAX_POS, NB), jnp.int32),
            pltpu.VMEM((MAX_POS, HIDDEN), jnp.float32),
            pltpu.VMEM((NB, HIDDEN), jnp.float32),
            pltpu.VMEM((NB, HIDDEN), jnp.float32),
            pltpu.SemaphoreType.DMA,
            pltpu.SemaphoreType.DMA,
        ],
    )
    def k(ids_hbm, table_hbm, pe_hbm, out_hbm, idx_v, pe_v, rows0, rows1,
          sem0, sem1):
        wid = lax.axis_index("s") * NC + lax.axis_index("c")
        base = wid * NB
        pltpu.sync_copy(ids_hbm.at[wid], idx_v)
        pltpu.sync_copy(pe_hbm, pe_v)
        rows = (rows0, rows1)
        sems = (sem0, sem1)

        pltpu.async_copy(table_hbm.at[idx_v.at[0]], rows[0], sems[0])

        def pair_body(i, carry):
            for b in range(2):
                s = 2 * i + b
                other = 1 - b

                @pl.when(s < MAX_POS)
                def _():
                    pltpu.make_async_copy(
                        table_hbm.at[idx_v.at[s]], rows[b], sems[b]).wait()

                    @pl.when(s + 1 < MAX_POS)
                    def _():
                        pltpu.async_copy(
                            table_hbm.at[idx_v.at[s + 1]], rows[other],
                            sems[other])

                    pes = [pe_v[s, pl.ds(j * LANES, LANES)]
                           for j in range(HIDDEN // LANES)]

                    @plsc.parallel_loop(0, NB, step=1, unroll=4)
                    def _(r):
                        for j in range(HIDDEN // LANES):
                            plsc.addupdate(
                                rows[b].at[r, pl.ds(j * LANES, LANES)],
                                pes[j])

                    pltpu.sync_copy(rows[b], out_hbm.at[s, pl.ds(base, NB)])
            return carry

        lax.fori_loop(0, (MAX_POS + 1) // 2, pair_body, 0, unroll=1)

    return k(ids3, table, pe)


def kernel(input_ids, token_table, pos_emb):
    batch, seq = input_ids.shape
    ids3 = (input_ids.astype(jnp.int32).T
            .reshape(seq, NW, NB).transpose(1, 0, 2))
    pe = pos_emb.astype(jnp.float32).reshape(MAX_POS, HIDDEN)
    out = _sc_embed(ids3, token_table.astype(jnp.float32), pe)
    return out.transpose(1, 0, 2)


# speedup vs baseline: 3.2367x; 1.1576x over previous
import functools

import jax
import jax.numpy as jnp
from jax import lax
from jax.experimental import pallas as pl
from jax.experimental.pallas import tpu as pltpu, tpu_sc as plsc

VOCAB = 49408
HIDDEN = 512
MAX_POS = 77
NC = 2
NS = 16
NW = NC * NS
LANES = 16
NB = 32
NBUF = 4


def _sc_embed(ids3, table, pe):
    mesh = plsc.VectorSubcoreMesh(core_axis_name="c", subcore_axis_name="s")

    @functools.partial(
        pl.kernel,
        mesh=mesh,
        out_type=jax.ShapeDtypeStruct(
            (MAX_POS, NW * NB, HIDDEN), jnp.float32),
        scratch_types=(
            [pltpu.VMEM((MAX_POS, NB), jnp.int32),
             pltpu.VMEM((MAX_POS, HIDDEN), jnp.float32)]
            + [pltpu.VMEM((NB, HIDDEN), jnp.float32)] * NBUF
            + [pltpu.SemaphoreType.DMA] * (2 * NBUF)
        ),
    )
    def k(ids_hbm, table_hbm, pe_hbm, out_hbm, idx_v, pe_v, *bufs):
        rows = bufs[:NBUF]
        gsems = bufs[NBUF:2 * NBUF]
        ssems = bufs[2 * NBUF:]
        wid = lax.axis_index("s") * NC + lax.axis_index("c")
        base = wid * NB
        pltpu.sync_copy(ids_hbm.at[wid], idx_v)
        pltpu.sync_copy(pe_hbm, pe_v)

        pltpu.async_copy(table_hbm.at[idx_v.at[0]], rows[0], gsems[0])
        pltpu.async_copy(table_hbm.at[idx_v.at[1]], rows[1], gsems[1])

        def quad_body(i, carry):
            for b in range(NBUF):
                s = NBUF * i + b

                @pl.when(s < MAX_POS)
                def _():
                    pltpu.make_async_copy(
                        table_hbm.at[idx_v.at[s]], rows[b], gsems[b]).wait()

                    b2 = (b + 2) % NBUF

                    @pl.when(s + 2 < MAX_POS)
                    def _():
                        @pl.when(s >= 2)
                        def _():
                            pltpu.make_async_copy(
                                rows[b2],
                                out_hbm.at[s - 2, pl.ds(base, NB)],
                                ssems[b2]).wait()

                        pltpu.async_copy(
                            table_hbm.at[idx_v.at[s + 2]], rows[b2],
                            gsems[b2])

                    pes = [pe_v[s, pl.ds(j * LANES, LANES)]
                           for j in range(HIDDEN // LANES)]

                    @plsc.parallel_loop(0, NB, step=1, unroll=4)
                    def _(r):
                        for j in range(HIDDEN // LANES):
                            plsc.addupdate(
                                rows[b].at[r, pl.ds(j * LANES, LANES)],
                                pes[j])

                    pltpu.async_copy(
                        rows[b], out_hbm.at[s, pl.ds(base, NB)], ssems[b])
            return carry

        lax.fori_loop(0, (MAX_POS + NBUF - 1) // NBUF, quad_body, 0,
                      unroll=1)

        for s in range(MAX_POS - 4, MAX_POS):
            pltpu.make_async_copy(
                rows[s % NBUF], out_hbm.at[s, pl.ds(base, NB)],
                ssems[s % NBUF]).wait()

    return k(ids3, table, pe)


def kernel(input_ids, token_table, pos_emb):
    batch, seq = input_ids.shape
    ids3 = (input_ids.astype(jnp.int32).T
            .reshape(seq, NW, NB).transpose(1, 0, 2))
    pe = pos_emb.astype(jnp.float32).reshape(MAX_POS, HIDDEN)
    out = _sc_embed(ids3, token_table.astype(jnp.float32), pe)
    return out.transpose(1, 0, 2)


# speedup vs baseline: 3.2592x; 1.0070x over previous
import functools

import jax
import jax.numpy as jnp
from jax import lax
from jax.experimental import pallas as pl
from jax.experimental.pallas import tpu as pltpu, tpu_sc as plsc

VOCAB = 49408
HIDDEN = 512
MAX_POS = 77
NC = 2
NS = 16
NW = NC * NS
LANES = 16
NB = 32
NBUF = 4


def _sc_embed(ids3, table, pe):
    mesh = plsc.VectorSubcoreMesh(core_axis_name="c", subcore_axis_name="s")

    @functools.partial(
        pl.kernel,
        mesh=mesh,
        out_type=jax.ShapeDtypeStruct(
            (MAX_POS, NW * NB, HIDDEN), jnp.float32),
        scratch_types=(
            [pltpu.VMEM((MAX_POS, NB), jnp.int32),
             pltpu.VMEM((MAX_POS, HIDDEN), jnp.float32)]
            + [pltpu.VMEM((NB, HIDDEN), jnp.float32)] * NBUF
            + [pltpu.SemaphoreType.DMA] * (2 * NBUF)
        ),
    )
    def k(ids_hbm, table_hbm, pe_hbm, out_hbm, idx_v, pe_v, *bufs):
        rows = bufs[:NBUF]
        gsems = bufs[NBUF:2 * NBUF]
        ssems = bufs[2 * NBUF:]
        wid = lax.axis_index("s") * NC + lax.axis_index("c")
        base = wid * NB
        pltpu.sync_copy(ids_hbm.at[wid], idx_v)
        pltpu.async_copy(table_hbm.at[idx_v.at[0]], rows[0], gsems[0])
        pltpu.async_copy(table_hbm.at[idx_v.at[1]], rows[1], gsems[1])
        pltpu.sync_copy(pe_hbm, pe_v)

        def quad_body(i, carry):
            for b in range(NBUF):
                s = NBUF * i + b

                @pl.when(s < MAX_POS)
                def _():
                    pltpu.make_async_copy(
                        table_hbm.at[idx_v.at[s]], rows[b], gsems[b]).wait()

                    b2 = (b + 2) % NBUF

                    @pl.when(s + 2 < MAX_POS)
                    def _():
                        @pl.when(s >= 2)
                        def _():
                            pltpu.make_async_copy(
                                rows[b2],
                                out_hbm.at[s - 2, pl.ds(base, NB)],
                                ssems[b2]).wait()

                        pltpu.async_copy(
                            table_hbm.at[idx_v.at[s + 2]], rows[b2],
                            gsems[b2])

                    pes = [pe_v[s, pl.ds(j * LANES, LANES)]
                           for j in range(HIDDEN // LANES)]

                    @plsc.parallel_loop(0, NB, step=1, unroll=4)
                    def _(r):
                        for j in range(HIDDEN // LANES):
                            plsc.addupdate(
                                rows[b].at[r, pl.ds(j * LANES, LANES)],
                                pes[j])

                    pltpu.async_copy(
                        rows[b], out_hbm.at[s, pl.ds(base, NB)], ssems[b])
            return carry

        lax.fori_loop(0, (MAX_POS + NBUF - 1) // NBUF, quad_body, 0,
                      unroll=1)

        for s in range(MAX_POS - 4, MAX_POS):
            pltpu.make_async_copy(
                rows[s % NBUF], out_hbm.at[s, pl.ds(base, NB)],
                ssems[s % NBUF]).wait()

    return k(ids3, table, pe)


def kernel(input_ids, token_table, pos_emb):
    batch, seq = input_ids.shape
    ids3 = (input_ids.astype(jnp.int32).T
            .reshape(seq, NW, NB).transpose(1, 0, 2))
    pe = pos_emb.astype(jnp.float32).reshape(MAX_POS, HIDDEN)
    out = _sc_embed(ids3, token_table.astype(jnp.float32), pe)
    return out.transpose(1, 0, 2)
